# baseline (speedup 1.0000x reference)
import functools

import jax
import jax.numpy as jnp
from jax.experimental import pallas as pl
from jax.experimental.pallas import tpu as pltpu

NUM_LANG = 64
IN_DIM = 1024
HID_DIM = 256
OUT_DIM = 1024
SEQ = 8
BZ = 128

BCOLS = 8
NB = BZ // BCOLS
NITEMS = NUM_LANG + NB - 1


def _routing(lang_ids):
    lang = lang_ids.astype(jnp.int32)
    t = jnp.arange(BZ, dtype=jnp.int32)
    e = jnp.arange(NUM_LANG, dtype=jnp.int32)
    Mi = (lang[None, :] == e[:, None]).astype(jnp.int32)
    counts = Mi.sum(axis=1)
    starts_e = jnp.cumsum(counts) - counts
    rank = (Mi * jnp.cumsum(Mi, axis=1)).sum(axis=0) - 1
    pos = (Mi * starts_e[:, None]).sum(axis=0) + rank
    Pi = (pos[None, :] == t[:, None]).astype(jnp.int32)
    perm = (Pi * t[None, :]).sum(axis=1)
    slang = (Pi * lang[None, :]).sum(axis=1)
    prev = jnp.concatenate([jnp.full((1,), -1, jnp.int32), slang[:-1]])
    starts_t = ((t % BCOLS) == 0) | (slang != prev)
    item_id = jnp.cumsum(starts_t.astype(jnp.int32)) - 1
    ii = jnp.arange(NITEMS, dtype=jnp.int32)
    I = item_id[None, :] == ii[:, None]
    t0 = jnp.min(jnp.where(I, t, BZ), axis=1)
    t1 = jnp.max(jnp.where(I, t + 1, 0), axis=1)
    num_items = item_id[BZ - 1] + 1
    last_t0 = jnp.sum(jnp.where(ii == num_items - 1, t0, 0))
    valid = ii < num_items
    t0 = jnp.where(valid, t0, last_t0)
    t1 = jnp.where(valid, t1, last_t0)
    tc = jnp.clip(t0, 0, BZ - 1)
    item_expert = ((tc[:, None] == t[None, :]) * slang[None, :]).sum(axis=1)
    item_block = t0 // BCOLS
    item_lo = t0 % BCOLS
    item_hi = t1 - item_block * BCOLS
    return perm, item_expert, item_block, item_lo, item_hi


def _mlp_body(expert_ref, block_ref, lo_ref, hi_ref, perm_ref,
              xt_ref, w1_ref, w2_ref, b1_ref, b2_ref, yt_ref, xg_ref):
    i = pl.program_id(0)
    lo = lo_ref[i]
    hi = hi_ref[i]
    blk = block_ref[i]
    e = expert_ref[i]

    @pl.when(hi > lo)
    def _():
        for j in range(BCOLS):
            col = perm_ref[blk * BCOLS + j]
            xg_ref[pl.ds(j * SEQ, SEQ), :] = xt_ref[col]
        xg = xg_ref[...]
        w1 = w1_ref[0]
        h = jax.lax.dot_general(xg, w1, (((1,), (1,)), ((), ())),
                                preferred_element_type=jnp.float32)
        h = jnp.maximum(h + b1_ref[e], 0.0)
        w2 = w2_ref[0]
        y = jax.lax.dot_general(h.astype(jnp.bfloat16), w2,
                                (((1,), (1,)), ((), ())),
                                preferred_element_type=jnp.float32)
        y = y + b2_ref[e]
        yb = y.reshape(BCOLS, SEQ, OUT_DIM)
        for j in range(BCOLS):
            @pl.when((j >= lo) & (j < hi))
            def _(j=j):
                col = perm_ref[blk * BCOLS + j]
                yt_ref[col] = yb[j]


@jax.jit
def kernel(x, lang_ids, W1, b1, W2, b2):
    perm, item_expert, item_block, item_lo, item_hi = _routing(lang_ids)
    xt = jnp.transpose(x, (1, 0, 2)).astype(jnp.bfloat16)
    W1 = W1.astype(jnp.bfloat16)
    W2 = W2.astype(jnp.bfloat16)

    grid_spec = pltpu.PrefetchScalarGridSpec(
        num_scalar_prefetch=5,
        grid=(NITEMS,),
        in_specs=[
            pl.BlockSpec((BZ, SEQ, IN_DIM), lambda i, *_: (0, 0, 0)),
            pl.BlockSpec((1, HID_DIM, IN_DIM),
                         lambda i, e_ref, *_: (e_ref[i], 0, 0)),
            pl.BlockSpec((1, OUT_DIM, HID_DIM),
                         lambda i, e_ref, *_: (e_ref[i], 0, 0)),
            pl.BlockSpec((NUM_LANG, HID_DIM), lambda i, *_: (0, 0)),
            pl.BlockSpec((NUM_LANG, OUT_DIM), lambda i, *_: (0, 0)),
        ],
        out_specs=pl.BlockSpec((BZ, SEQ, OUT_DIM), lambda i, *_: (0, 0, 0)),
        scratch_shapes=[pltpu.VMEM((BCOLS * SEQ, IN_DIM), jnp.bfloat16)],
    )
    yt = pl.pallas_call(
        _mlp_body,
        grid_spec=grid_spec,
        out_shape=jax.ShapeDtypeStruct((BZ, SEQ, OUT_DIM), jnp.float32),
    )(item_expert, item_block, item_lo, item_hi, perm,
      xt, W1, W2, b1, b2)
    return jnp.transpose(yt, (1, 0, 2))


# speedup vs baseline: 1.6890x; 1.6890x over previous
import functools

import jax
import jax.numpy as jnp
from jax.experimental import pallas as pl
from jax.experimental.pallas import tpu as pltpu

NUM_LANG = 64
IN_DIM = 1024
HID_DIM = 256
OUT_DIM = 1024
SEQ = 8
BZ = 128

BCOLS = 8
NB = BZ // BCOLS
NITEMS = NUM_LANG + NB - 1


def _routing(lang_ids):
    lang = lang_ids.astype(jnp.int32)
    t = jnp.arange(BZ, dtype=jnp.int32)
    e = jnp.arange(NUM_LANG, dtype=jnp.int32)
    Mi = (lang[None, :] == e[:, None]).astype(jnp.int32)
    counts = Mi.sum(axis=1)
    starts_e = jnp.cumsum(counts) - counts
    rank = (Mi * jnp.cumsum(Mi, axis=1)).sum(axis=0) - 1
    pos = (Mi * starts_e[:, None]).sum(axis=0) + rank
    Pi = (pos[None, :] == t[:, None]).astype(jnp.int32)
    perm = (Pi * t[None, :]).sum(axis=1)
    slang = (Pi * lang[None, :]).sum(axis=1)
    prev = jnp.concatenate([jnp.full((1,), -1, jnp.int32), slang[:-1]])
    starts_t = ((t % BCOLS) == 0) | (slang != prev)
    item_id = jnp.cumsum(starts_t.astype(jnp.int32)) - 1
    ii = jnp.arange(NITEMS, dtype=jnp.int32)
    I = item_id[None, :] == ii[:, None]
    t0 = jnp.min(jnp.where(I, t, BZ), axis=1)
    t1 = jnp.max(jnp.where(I, t + 1, 0), axis=1)
    num_items = item_id[BZ - 1] + 1
    last_t0 = jnp.sum(jnp.where(ii == num_items - 1, t0, 0))
    valid = ii < num_items
    t0 = jnp.where(valid, t0, last_t0)
    t1 = jnp.where(valid, t1, last_t0)
    tc = jnp.clip(t0, 0, BZ - 1)
    item_expert = ((tc[:, None] == t[None, :]) * slang[None, :]).sum(axis=1)
    item_block = t0 // BCOLS
    item_lo = t0 % BCOLS
    item_hi = t1 - item_block * BCOLS
    return perm, item_expert, item_block, item_lo, item_hi


def _mlp_body(expert_ref, block_ref, lo_ref, hi_ref, perm_ref,
              x_ref, w1_ref, w2_ref, b1_ref, b2_ref, y_ref, xg_ref):
    i = pl.program_id(0)
    lo = lo_ref[i]
    hi = hi_ref[i]
    blk = block_ref[i]
    e = expert_ref[i]

    @pl.when(hi > lo)
    def _():
        for j in range(BCOLS):
            col = perm_ref[blk * BCOLS + j]
            xg_ref[pl.ds(j * SEQ, SEQ), :] = x_ref[:, col, :]
        xg = xg_ref[...]
        w1 = w1_ref[0]
        h = jax.lax.dot_general(xg, w1, (((1,), (1,)), ((), ())),
                                preferred_element_type=jnp.float32)
        h = jnp.maximum(h + b1_ref[e], 0.0)
        w2 = w2_ref[0]
        y = jax.lax.dot_general(h, w2, (((1,), (1,)), ((), ())),
                                preferred_element_type=jnp.float32)
        y = y + b2_ref[e]
        yb = y.reshape(BCOLS, SEQ, OUT_DIM)
        for j in range(BCOLS):
            @pl.when((j >= lo) & (j < hi))
            def _(j=j):
                col = perm_ref[blk * BCOLS + j]
                y_ref[:, col, :] = yb[j]


@jax.jit
def kernel(x, lang_ids, W1, b1, W2, b2):
    perm, item_expert, item_block, item_lo, item_hi = _routing(lang_ids)

    grid_spec = pltpu.PrefetchScalarGridSpec(
        num_scalar_prefetch=5,
        grid=(NITEMS,),
        in_specs=[
            pl.BlockSpec((SEQ, BZ, IN_DIM), lambda i, *_: (0, 0, 0)),
            pl.BlockSpec((1, HID_DIM, IN_DIM),
                         lambda i, e_ref, *_: (e_ref[i], 0, 0)),
            pl.BlockSpec((1, OUT_DIM, HID_DIM),
                         lambda i, e_ref, *_: (e_ref[i], 0, 0)),
            pl.BlockSpec((NUM_LANG, HID_DIM), lambda i, *_: (0, 0)),
            pl.BlockSpec((NUM_LANG, OUT_DIM), lambda i, *_: (0, 0)),
        ],
        out_specs=pl.BlockSpec((SEQ, BZ, OUT_DIM), lambda i, *_: (0, 0, 0)),
        scratch_shapes=[pltpu.VMEM((BCOLS * SEQ, IN_DIM), jnp.float32)],
    )
    y = pl.pallas_call(
        _mlp_body,
        grid_spec=grid_spec,
        out_shape=jax.ShapeDtypeStruct((SEQ, BZ, OUT_DIM), jnp.float32),
    )(item_expert, item_block, item_lo, item_hi, perm,
      x, W1, W2, b1, b2)
    return y


# speedup vs baseline: 1.7881x; 1.0586x over previous
import functools

import jax
import jax.numpy as jnp
from jax.experimental import pallas as pl
from jax.experimental.pallas import tpu as pltpu

NUM_LANG = 64
IN_DIM = 1024
HID_DIM = 256
OUT_DIM = 1024
SEQ = 8
BZ = 128

BCOLS = 8
NB = BZ // BCOLS
NITEMS = NUM_LANG + NB - 1


def _routing(lang_ids):
    lang = lang_ids.astype(jnp.int32)
    t = jnp.arange(BZ, dtype=jnp.int32)
    e = jnp.arange(NUM_LANG, dtype=jnp.int32)
    Mi = (lang[None, :] == e[:, None]).astype(jnp.int32)
    counts = Mi.sum(axis=1)
    starts_e = jnp.cumsum(counts) - counts
    rank = (Mi * jnp.cumsum(Mi, axis=1)).sum(axis=0) - 1
    pos = (Mi * starts_e[:, None]).sum(axis=0) + rank
    Pi = (pos[None, :] == t[:, None]).astype(jnp.int32)
    perm = (Pi * t[None, :]).sum(axis=1)
    slang = (Pi * lang[None, :]).sum(axis=1)
    prev = jnp.concatenate([jnp.full((1,), -1, jnp.int32), slang[:-1]])
    starts_t = ((t % BCOLS) == 0) | (slang != prev)
    item_id = jnp.cumsum(starts_t.astype(jnp.int32)) - 1
    ii = jnp.arange(NITEMS, dtype=jnp.int32)
    I = item_id[None, :] == ii[:, None]
    t0 = jnp.min(jnp.where(I, t, BZ), axis=1)
    t1 = jnp.max(jnp.where(I, t + 1, 0), axis=1)
    num_items = item_id[BZ - 1] + 1
    last_t0 = jnp.sum(jnp.where(ii == num_items - 1, t0, 0))
    valid = ii < num_items
    t0 = jnp.where(valid, t0, last_t0)
    t1 = jnp.where(valid, t1, last_t0)
    tc = jnp.clip(t0, 0, BZ - 1)
    item_expert = ((tc[:, None] == t[None, :]) * slang[None, :]).sum(axis=1)
    item_block = t0 // BCOLS
    item_lo = t0 % BCOLS
    item_hi = t1 - item_block * BCOLS
    return perm, item_expert, item_block, item_lo, item_hi


def _mlp_body(expert_ref, block_ref, lo_ref, hi_ref, perm_ref,
              x_ref, w1a_ref, w1b_ref, w2a_ref, w2b_ref, b1_ref, b2_ref,
              y_ref, xg_ref):
    i = pl.program_id(0)
    lo = lo_ref[i]
    hi = hi_ref[i]
    blk = block_ref[i]
    e = expert_ref[i]

    @pl.when(hi > lo)
    def _():
        for j in range(BCOLS):
            col = perm_ref[blk * BCOLS + j]
            xg_ref[pl.ds(j * SEQ, SEQ), :] = x_ref[:, col, :]
        xg = xg_ref[...]
        dn = (((1,), (1,)), ((), ()))
        ha = jax.lax.dot_general(xg, w1a_ref[0], dn,
                                 preferred_element_type=jnp.float32)
        hb = jax.lax.dot_general(xg, w1b_ref[0], dn,
                                 preferred_element_type=jnp.float32)
        h = jnp.concatenate([ha, hb], axis=1)
        h = jnp.maximum(h + b1_ref[e], 0.0)
        ya = jax.lax.dot_general(h, w2a_ref[0], dn,
                                 preferred_element_type=jnp.float32)
        yb2 = jax.lax.dot_general(h, w2b_ref[0], dn,
                                  preferred_element_type=jnp.float32)
        y = jnp.concatenate([ya, yb2], axis=1)
        y = y + b2_ref[e]
        yb = y.reshape(BCOLS, SEQ, OUT_DIM)
        for j in range(BCOLS):
            @pl.when((j >= lo) & (j < hi))
            def _(j=j):
                col = perm_ref[blk * BCOLS + j]
                y_ref[:, col, :] = yb[j]


@jax.jit
def kernel(x, lang_ids, W1, b1, W2, b2):
    perm, item_expert, item_block, item_lo, item_hi = _routing(lang_ids)

    grid_spec = pltpu.PrefetchScalarGridSpec(
        num_scalar_prefetch=5,
        grid=(NITEMS,),
        in_specs=[
            pl.BlockSpec((SEQ, BZ, IN_DIM), lambda i, *_: (0, 0, 0)),
            pl.BlockSpec((1, HID_DIM // 2, IN_DIM),
                         lambda i, e_ref, *_: (e_ref[i], 0, 0)),
            pl.BlockSpec((1, HID_DIM // 2, IN_DIM),
                         lambda i, e_ref, *_: (e_ref[i], 1, 0)),
            pl.BlockSpec((1, OUT_DIM // 2, HID_DIM),
                         lambda i, e_ref, *_: (e_ref[i], 0, 0)),
            pl.BlockSpec((1, OUT_DIM // 2, HID_DIM),
                         lambda i, e_ref, *_: (e_ref[i], 1, 0)),
            pl.BlockSpec((NUM_LANG, HID_DIM), lambda i, *_: (0, 0)),
            pl.BlockSpec((NUM_LANG, OUT_DIM), lambda i, *_: (0, 0)),
        ],
        out_specs=pl.BlockSpec((SEQ, BZ, OUT_DIM), lambda i, *_: (0, 0, 0)),
        scratch_shapes=[pltpu.VMEM((BCOLS * SEQ, IN_DIM), jnp.float32)],
    )
    y = pl.pallas_call(
        _mlp_body,
        grid_spec=grid_spec,
        out_shape=jax.ShapeDtypeStruct((SEQ, BZ, OUT_DIM), jnp.float32),
    )(item_expert, item_block, item_lo, item_hi, perm,
      x, W1, W1, W2, W2, b1, b2)
    return y
